# baseline (speedup 1.0000x reference)
import math

import jax
import jax.numpy as jnp
from jax import lax
from jax.experimental import pallas as pl
from jax.experimental.pallas import tpu as pltpu
from jax.experimental.pallas import tpu_sc as plsc

_EM = math.exp(-1e-6 - 1.0)

_P = 1024
_M = 64
_NC = 2
_NS = 16
_NW = _NC * _NS
_ROWS = _P // _NW
_L = 16
_PR = 2
_MC = _M // _L
_SUB = 2


def _sc_body(hid_hbm, nei_hbm, out_hbm, cnt_hbm, hid_v, nei_v, out_v, cnt_v):
    wid = lax.axis_index("s") * _NC + lax.axis_index("c")
    base = wid * _ROWS
    pltpu.sync_copy(hid_hbm, hid_v)
    pltpu.sync_copy(nei_hbm.at[pl.ds(base * _P, _ROWS * _P)], nei_v)

    zero = jnp.zeros((_L,), jnp.float32)

    def group_body(rg, total):
        r0 = rg * _PR

        def chunk_body(jc, carry):
            accs, cnts = carry
            j0 = jc * _L
            nvs = [nei_v[pl.ds((r0 + q) * _P + j0, _L)] for q in range(_PR)]
            cnts = list(cnts)
            accs = [[list(sub) for sub in row] for row in accs]
            for l in range(_L):
                h0 = (j0 + l) * _M
                hs = [hid_v[pl.ds(h0 + c * _L, _L)] for c in range(_MC)]
                u = l % _SUB
                for q in range(_PR):
                    pred = nvs[q][l] > 0
                    for c in range(_MC):
                        accs[q][c][u] = jnp.where(
                            pred, accs[q][c][u] + hs[c], accs[q][c][u])
                    cnts[q] = jnp.where(pred, cnts[q] + 1.0, cnts[q])
            return (tuple(tuple(tuple(sub) for sub in row) for row in accs),
                    tuple(cnts))

        acc0 = tuple(tuple(tuple(zero for _ in range(_SUB))
                           for _ in range(_MC)) for _ in range(_PR))
        cnt0 = tuple(jnp.float32(0.0) for _ in range(_PR))
        accs, cnts = lax.fori_loop(0, _P // _L, chunk_body, (acc0, cnt0))

        for q in range(_PR):
            k = cnts[q]
            den = k + (_P - k) * _EM
            scale = 1.0 / jnp.full((_L,), den, jnp.float32)
            o0 = (r0 + q) * _M
            for c in range(_MC):
                acc = accs[q][c][0]
                for u in range(1, _SUB):
                    acc = acc + accs[q][c][u]
                out_v[pl.ds(o0 + c * _L, _L)] = acc * scale
            total = total + k
        return total

    total = lax.fori_loop(0, _ROWS // _PR, group_body,
                          jnp.zeros((_L,), jnp.float32))
    cnt_v[pl.ds(0, _L)] = total
    pltpu.sync_copy(out_v, out_hbm.at[pl.ds(base * _M, _ROWS * _M)])
    pltpu.sync_copy(cnt_v, cnt_hbm.at[pl.ds(wid * _L, _L)])


_sc_call = pl.kernel(
    _sc_body,
    out_type=(
        jax.ShapeDtypeStruct((_P * _M,), jnp.float32),
        jax.ShapeDtypeStruct((_NW * _L,), jnp.float32),
    ),
    mesh=plsc.VectorSubcoreMesh(core_axis_name="c", subcore_axis_name="s"),
    scratch_types=[
        pltpu.VMEM((_P * _M,), jnp.float32),
        pltpu.VMEM((_ROWS * _P,), jnp.int32),
        pltpu.VMEM((_ROWS * _M,), jnp.float32),
        pltpu.VMEM((_L,), jnp.float32),
    ],
)


def kernel(hidden_state, corr_index, nei_index):
    del corr_index
    out, cnt = _sc_call(hidden_state.reshape(-1), nei_index.reshape(-1))
    has = jnp.any(cnt > 0.0)
    return jnp.where(has, out.reshape(_P, _M), hidden_state)


# speedup vs baseline: 2.7725x; 2.7725x over previous
import math

import jax
import jax.numpy as jnp
from jax import lax
from jax.experimental import pallas as pl
from jax.experimental.pallas import tpu as pltpu
from jax.experimental.pallas import tpu_sc as plsc

_EM = math.exp(-1e-6 - 1.0)

_P = 1024
_M = 64
_NC = 2
_NS = 16
_NW = _NC * _NS
_SC_ROWS = 64
_ROWS = _SC_ROWS // _NW
_TC_ROWS = _P - _SC_ROWS
_L = 16
_MC = _M // _L


def _sc_body(hid_hbm, nei_hbm, out_hbm, cnt_hbm, hid_v, nei_v, out_v, cnt_v):
    wid = lax.axis_index("s") * _NC + lax.axis_index("c")
    base = wid * _ROWS
    pltpu.sync_copy(hid_hbm, hid_v)
    pltpu.sync_copy(nei_hbm.at[pl.ds(base * _P, _ROWS * _P)], nei_v)

    zero = jnp.zeros((_L,), jnp.float32)

    def row_body(r, total):
        def chunk_body(jc, carry):
            a0, a1, a2, a3, cnt = carry
            j0 = jc * _L
            nv = nei_v[pl.ds(r * _P + j0, _L)]
            for l in range(_L):
                h0 = (j0 + l) * _M
                pred = nv[l] > 0
                a0 = jnp.where(pred, a0 + hid_v[pl.ds(h0, _L)], a0)
                a1 = jnp.where(pred, a1 + hid_v[pl.ds(h0 + _L, _L)], a1)
                a2 = jnp.where(pred, a2 + hid_v[pl.ds(h0 + 2 * _L, _L)], a2)
                a3 = jnp.where(pred, a3 + hid_v[pl.ds(h0 + 3 * _L, _L)], a3)
                cnt = jnp.where(pred, cnt + 1.0, cnt)
            return a0, a1, a2, a3, cnt

        a0, a1, a2, a3, cnt = lax.fori_loop(
            0, _P // _L, chunk_body,
            (zero, zero, zero, zero, jnp.float32(0.0)))
        den = cnt + (_P - cnt) * _EM
        scale = 1.0 / jnp.full((_L,), den, jnp.float32)
        o0 = r * _M
        out_v[pl.ds(o0, _L)] = a0 * scale
        out_v[pl.ds(o0 + _L, _L)] = a1 * scale
        out_v[pl.ds(o0 + 2 * _L, _L)] = a2 * scale
        out_v[pl.ds(o0 + 3 * _L, _L)] = a3 * scale
        return total + cnt

    total = lax.fori_loop(0, _ROWS, row_body, jnp.zeros((_L,), jnp.float32))
    cnt_v[pl.ds(0, _L)] = total
    pltpu.sync_copy(out_v, out_hbm.at[pl.ds(base * _M, _ROWS * _M)])
    pltpu.sync_copy(cnt_v, cnt_hbm.at[pl.ds(wid * _L, _L)])


_sc_call = pl.kernel(
    _sc_body,
    out_type=(
        jax.ShapeDtypeStruct((_SC_ROWS * _M,), jnp.float32),
        jax.ShapeDtypeStruct((_NW * _L,), jnp.float32),
    ),
    mesh=plsc.VectorSubcoreMesh(core_axis_name="c", subcore_axis_name="s"),
    scratch_types=[
        pltpu.VMEM((_P * _M,), jnp.float32),
        pltpu.VMEM((_ROWS * _P,), jnp.int32),
        pltpu.VMEM((_ROWS * _M,), jnp.float32),
        pltpu.VMEM((_L,), jnp.float32),
    ],
)


def _tc_body(hs_ref, nei_ref, out_ref, k_ref):
    mask = nei_ref[_SC_ROWS:, :] > 0
    mf = mask.astype(jnp.float32)
    k = jnp.sum(mf, axis=1, keepdims=True)
    scale = 1.0 / (k + (_P - k) * _EM)
    acc = jnp.dot(mf, hs_ref[...], preferred_element_type=jnp.float32)
    out_ref[...] = scale * acc
    k_ref[...] = k


def _tc_call(hidden_state, nei_index):
    return pl.pallas_call(
        _tc_body,
        out_shape=(
            jax.ShapeDtypeStruct((_TC_ROWS, _M), jnp.float32),
            jax.ShapeDtypeStruct((_TC_ROWS, 1), jnp.float32),
        ),
    )(hidden_state, nei_index)


def kernel(hidden_state, corr_index, nei_index):
    del corr_index
    sc_out, sc_cnt = _sc_call(hidden_state.reshape(-1), nei_index.reshape(-1))
    tc_out, tc_k = _tc_call(hidden_state, nei_index)
    out = jnp.concatenate([sc_out.reshape(_SC_ROWS, _M), tc_out], axis=0)
    has = jnp.any(sc_cnt > 0.0) | jnp.any(tc_k > 0.0)
    return jnp.where(has, out, hidden_state)


# speedup vs baseline: 3.3810x; 1.2195x over previous
import math

import jax
import jax.numpy as jnp
from jax import lax
from jax.experimental import pallas as pl
from jax.experimental.pallas import tpu as pltpu
from jax.experimental.pallas import tpu_sc as plsc

_EM = math.exp(-1e-6 - 1.0)

_P = 1024
_M = 64
_NC = 2
_NS = 16
_NW = _NC * _NS
_ROWS = _P // _NW
_L = 16


def _sc_body(nei_hbm, cnt_hbm, nei_v, cnt_v):
    wid = lax.axis_index("s") * _NC + lax.axis_index("c")
    base = wid * _ROWS
    pltpu.sync_copy(nei_hbm.at[pl.ds(base * _P, _ROWS * _P)], nei_v)

    zero = jnp.zeros((_L,), jnp.float32)
    one = jnp.full((_L,), 1.0, jnp.float32)

    def row_body(r, _):
        def chunk_body(jc, cv):
            nv = nei_v[pl.ds(r * _P + jc * _L, _L)]
            return cv + jnp.where(nv > 0, one, zero)

        cv = lax.fori_loop(0, _P // _L, chunk_body, zero)
        cnt_v[pl.ds(r * _L, _L)] = cv
        return 0

    lax.fori_loop(0, _ROWS, row_body, 0)
    pltpu.sync_copy(cnt_v, cnt_hbm.at[pl.ds(base * _L, _ROWS * _L)])


_sc_call = pl.kernel(
    _sc_body,
    out_type=jax.ShapeDtypeStruct((_P * _L,), jnp.float32),
    mesh=plsc.VectorSubcoreMesh(core_axis_name="c", subcore_axis_name="s"),
    scratch_types=[
        pltpu.VMEM((_ROWS * _P,), jnp.int32),
        pltpu.VMEM((_ROWS * _L,), jnp.float32),
    ],
)


def _tc_body(hs_ref, nei_ref, out_ref):
    mf = (nei_ref[...] > 0).astype(jnp.float32)
    out_ref[...] = jnp.dot(mf, hs_ref[...],
                           preferred_element_type=jnp.float32)


def _tc_call(hidden_state, nei_index):
    return pl.pallas_call(
        _tc_body,
        out_shape=jax.ShapeDtypeStruct((_P, _M), jnp.float32),
    )(hidden_state, nei_index)


def kernel(hidden_state, corr_index, nei_index):
    del corr_index
    lane_cnt = _sc_call(nei_index.reshape(-1))
    acc = _tc_call(hidden_state, nei_index)
    k = jnp.sum(lane_cnt.reshape(_P, _L), axis=1, keepdims=True)
    scale = 1.0 / (k + (_P - k) * _EM)
    has = jnp.any(k > 0.0)
    return jnp.where(has, acc * scale, hidden_state)
